# baseline (speedup 1.0000x reference)
import jax
import jax.numpy as jnp
from jax.experimental import pallas as pl
from jax.experimental.pallas import tpu as pltpu

_BQ = 256


def _encode_body(x_ref, bv_ref, dw_ref, en_ref, v2_ref):
    e = jnp.dot(x_ref[...], bv_ref[...], preferred_element_type=jnp.float32)
    norm = jnp.sqrt(jnp.sum(e * e, axis=-1, keepdims=True))
    en_ref[...] = (e / jnp.maximum(norm, 1e-8)).astype(jnp.bfloat16)
    v2_ref[...] = jax.lax.dot_general(
        e.astype(jnp.bfloat16), dw_ref[...],
        dimension_numbers=(((1,), (1,)), ((), ())),
        preferred_element_type=jnp.float32,
    ).astype(jnp.bfloat16)


def _attend_body(enq_ref, enk_ref, imp_ref, v2_ref, db_ref, x_ref, out_ref):
    s = jax.lax.dot_general(
        enq_ref[...], enk_ref[...],
        dimension_numbers=(((1,), (1,)), ((), ())),
        preferred_element_type=jnp.float32,
    )
    w = s * imp_ref[...]
    m = jnp.max(w, axis=-1, keepdims=True)
    p = jnp.exp(w - m)
    p = p / jnp.sum(p, axis=-1, keepdims=True)
    r = jnp.dot(p.astype(jnp.bfloat16), v2_ref[...], preferred_element_type=jnp.float32)
    out_ref[...] = r + db_ref[...] + x_ref[...]


def kernel(x, importance, base_vectors, dec_w, dec_b, memory_storage, memory_importance):
    Bx = x.shape[0]
    hidden = x.shape[2]
    hd = base_vectors.shape[1]
    x_flat = x.reshape(Bx, hidden)
    nblk = Bx // _BQ

    en, v2 = pl.pallas_call(
        _encode_body,
        grid=(nblk,),
        in_specs=[
            pl.BlockSpec((_BQ, hidden), lambda i: (i, 0)),
            pl.BlockSpec((hidden, hd), lambda i: (0, 0)),
            pl.BlockSpec((hidden, hd), lambda i: (0, 0)),
        ],
        out_specs=[
            pl.BlockSpec((_BQ, hd), lambda i: (i, 0)),
            pl.BlockSpec((_BQ, hidden), lambda i: (i, 0)),
        ],
        out_shape=[
            jax.ShapeDtypeStruct((Bx, hd), jnp.bfloat16),
            jax.ShapeDtypeStruct((Bx, hidden), jnp.bfloat16),
        ],
    )(x_flat.astype(jnp.bfloat16), base_vectors.astype(jnp.bfloat16),
      dec_w.astype(jnp.bfloat16))

    out = pl.pallas_call(
        _attend_body,
        grid=(nblk,),
        in_specs=[
            pl.BlockSpec((_BQ, hd), lambda i: (i, 0)),
            pl.BlockSpec((Bx, hd), lambda i: (0, 0)),
            pl.BlockSpec((1, Bx), lambda i: (0, 0)),
            pl.BlockSpec((Bx, hidden), lambda i: (0, 0)),
            pl.BlockSpec((1, hidden), lambda i: (0, 0)),
            pl.BlockSpec((_BQ, hidden), lambda i: (i, 0)),
        ],
        out_specs=pl.BlockSpec((_BQ, hidden), lambda i: (i, 0)),
        out_shape=jax.ShapeDtypeStruct((Bx, hidden), jnp.float32),
    )(en, en, importance.reshape(1, Bx), v2, dec_b.reshape(1, hidden), x_flat)

    return out.reshape(Bx, 1, hidden)


# speedup vs baseline: 1.1175x; 1.1175x over previous
import jax
import jax.numpy as jnp
from jax.experimental import pallas as pl
from jax.experimental.pallas import tpu as pltpu

_BQ = 256


def _fused_body(x_ref, bv_ref, dw_ref, imp_ref, db_ref, out_ref, en_sc, v2_sc):
    i = pl.program_id(0)
    nblk = pl.num_programs(0) // 2

    @pl.when(i < nblk)
    def _encode():
        e = jnp.dot(x_ref[...], bv_ref[...], preferred_element_type=jnp.float32)
        norm = jnp.sqrt(jnp.sum(e * e, axis=-1, keepdims=True))
        en_sc[pl.ds(i * _BQ, _BQ), :] = (e / jnp.maximum(norm, 1e-8)).astype(jnp.bfloat16)
        v2_sc[pl.ds(i * _BQ, _BQ), :] = jax.lax.dot_general(
            e, dw_ref[...],
            dimension_numbers=(((1,), (1,)), ((), ())),
            preferred_element_type=jnp.float32,
        ).astype(jnp.bfloat16)

    @pl.when(i >= nblk)
    def _attend():
        j = i - nblk
        enq = en_sc[pl.ds(j * _BQ, _BQ), :]
        s = jax.lax.dot_general(
            enq, en_sc[...],
            dimension_numbers=(((1,), (1,)), ((), ())),
            preferred_element_type=jnp.float32,
        )
        w = s * imp_ref[...]
        m = jnp.max(w, axis=-1, keepdims=True)
        p = jnp.exp(w - m)
        p = p / jnp.sum(p, axis=-1, keepdims=True)
        r = jnp.dot(p.astype(jnp.bfloat16), v2_sc[...], preferred_element_type=jnp.float32)
        out_ref[...] = r + db_ref[...] + x_ref[...]


def kernel(x, importance, base_vectors, dec_w, dec_b, memory_storage, memory_importance):
    Bx = x.shape[0]
    hidden = x.shape[2]
    hd = base_vectors.shape[1]
    x_flat = x.reshape(Bx, hidden)
    nblk = Bx // _BQ

    out = pl.pallas_call(
        _fused_body,
        grid=(2 * nblk,),
        in_specs=[
            pl.BlockSpec((_BQ, hidden), lambda i: (i % (pl.num_programs(0) // 2), 0)),
            pl.BlockSpec((hidden, hd), lambda i: (0, 0)),
            pl.BlockSpec((hidden, hd), lambda i: (0, 0)),
            pl.BlockSpec((1, Bx), lambda i: (0, 0)),
            pl.BlockSpec((1, hidden), lambda i: (0, 0)),
        ],
        out_specs=pl.BlockSpec(
            (_BQ, hidden),
            lambda i: (jnp.maximum(i - pl.num_programs(0) // 2, 0), 0),
        ),
        out_shape=jax.ShapeDtypeStruct((Bx, hidden), jnp.float32),
        scratch_shapes=[
            pltpu.VMEM((Bx, hd), jnp.bfloat16),
            pltpu.VMEM((Bx, hidden), jnp.bfloat16),
        ],
    )(x_flat, base_vectors, dec_w, importance.reshape(1, Bx), dec_b.reshape(1, hidden))

    return out.reshape(Bx, 1, hidden)


# speedup vs baseline: 1.7067x; 1.5272x over previous
import jax
import jax.numpy as jnp
from jax.experimental import pallas as pl
from jax.experimental.pallas import tpu as pltpu

_BQ = 256


def _fused_body(x_ref, bv_ref, dw_ref, imp_ref, db_ref, out_ref, en_sc, v2_sc):
    i = pl.program_id(0)
    nblk = pl.num_programs(0) // 2

    @pl.when(i < nblk)
    def _encode():
        e = jnp.dot(x_ref[:, 0, :], bv_ref[...], preferred_element_type=jnp.float32)
        norm = jnp.sqrt(jnp.sum(e * e, axis=-1, keepdims=True))
        en_sc[pl.ds(i * _BQ, _BQ), :] = (e / jnp.maximum(norm, 1e-8)).astype(jnp.bfloat16)
        v2_sc[pl.ds(i * _BQ, _BQ), :] = jax.lax.dot_general(
            e, dw_ref[...],
            dimension_numbers=(((1,), (1,)), ((), ())),
            preferred_element_type=jnp.float32,
        ).astype(jnp.bfloat16)

    @pl.when(i >= nblk)
    def _attend():
        j = i - nblk
        enq = en_sc[pl.ds(j * _BQ, _BQ), :]
        s = jax.lax.dot_general(
            enq, en_sc[...],
            dimension_numbers=(((1,), (1,)), ((), ())),
            preferred_element_type=jnp.float32,
        )
        w = s * imp_ref[...]
        m = jnp.max(w, axis=-1, keepdims=True)
        p = jnp.exp(w - m)
        p = p / jnp.sum(p, axis=-1, keepdims=True)
        r = jnp.dot(p.astype(jnp.bfloat16), v2_sc[...], preferred_element_type=jnp.float32)
        out_ref[:, 0, :] = r + db_ref[...] + x_ref[:, 0, :]


def kernel(x, importance, base_vectors, dec_w, dec_b, memory_storage, memory_importance):
    Bx = x.shape[0]
    hidden = x.shape[2]
    hd = base_vectors.shape[1]
    nblk = Bx // _BQ

    out = pl.pallas_call(
        _fused_body,
        grid=(2 * nblk,),
        in_specs=[
            pl.BlockSpec((_BQ, 1, hidden), lambda i: (i % (pl.num_programs(0) // 2), 0, 0)),
            pl.BlockSpec((hidden, hd), lambda i: (0, 0)),
            pl.BlockSpec((hidden, hd), lambda i: (0, 0)),
            pl.BlockSpec((1, Bx), lambda i: (0, 0)),
            pl.BlockSpec((1, hidden), lambda i: (0, 0)),
        ],
        out_specs=pl.BlockSpec(
            (_BQ, 1, hidden),
            lambda i: (jnp.maximum(i - pl.num_programs(0) // 2, 0), 0, 0),
        ),
        out_shape=jax.ShapeDtypeStruct((Bx, 1, hidden), jnp.float32),
        scratch_shapes=[
            pltpu.VMEM((Bx, hd), jnp.bfloat16),
            pltpu.VMEM((Bx, hidden), jnp.bfloat16),
        ],
    )(x, base_vectors, dec_w, importance.reshape(1, Bx), dec_b.reshape(1, hidden))

    return out


# speedup vs baseline: 1.7866x; 1.0468x over previous
import jax
import jax.numpy as jnp
from jax.experimental import pallas as pl
from jax.experimental.pallas import tpu as pltpu

_BQ = 256


def _fused_body(x_ref, bv_ref, dw_ref, imp_ref, db_ref, out_ref, en_sc, v2_sc):
    i = pl.program_id(0)
    nblk = pl.num_programs(0) // 2

    @pl.when(i < nblk)
    def _encode():
        e = jnp.dot(x_ref[:, 0, :], bv_ref[...], preferred_element_type=jnp.float32)
        norm = jnp.sqrt(jnp.sum(e * e, axis=-1, keepdims=True))
        en_sc[pl.ds(i * _BQ, _BQ), :] = (e / jnp.maximum(norm, 1e-8)).astype(jnp.bfloat16)
        v2_sc[pl.ds(i * _BQ, _BQ), :] = jax.lax.dot_general(
            e, dw_ref[...],
            dimension_numbers=(((1,), (1,)), ((), ())),
            preferred_element_type=jnp.float32,
        ).astype(jnp.bfloat16)

    @pl.when(i >= nblk)
    def _attend():
        j = i - nblk
        enq = en_sc[pl.ds(j * _BQ, _BQ), :]
        s = jax.lax.dot_general(
            enq, en_sc[...],
            dimension_numbers=(((1,), (1,)), ((), ())),
            preferred_element_type=jnp.float32,
        )
        p = jnp.exp(s * imp_ref[...])
        r = jnp.dot(p.astype(jnp.bfloat16), v2_sc[...], preferred_element_type=jnp.float32)
        denom = jnp.sum(p, axis=-1, keepdims=True)
        out_ref[:, 0, :] = r / denom + db_ref[...] + x_ref[:, 0, :]


def kernel(x, importance, base_vectors, dec_w, dec_b, memory_storage, memory_importance):
    Bx = x.shape[0]
    hidden = x.shape[2]
    hd = base_vectors.shape[1]
    nblk = Bx // _BQ

    out = pl.pallas_call(
        _fused_body,
        grid=(2 * nblk,),
        in_specs=[
            pl.BlockSpec((_BQ, 1, hidden), lambda i: (i % (pl.num_programs(0) // 2), 0, 0)),
            pl.BlockSpec((hidden, hd), lambda i: (0, 0)),
            pl.BlockSpec((hidden, hd), lambda i: (0, 0)),
            pl.BlockSpec((1, Bx), lambda i: (0, 0)),
            pl.BlockSpec((1, hidden), lambda i: (0, 0)),
        ],
        out_specs=pl.BlockSpec(
            (_BQ, 1, hidden),
            lambda i: (jnp.maximum(i - pl.num_programs(0) // 2, 0), 0, 0),
        ),
        out_shape=jax.ShapeDtypeStruct((Bx, 1, hidden), jnp.float32),
        scratch_shapes=[
            pltpu.VMEM((Bx, hd), jnp.bfloat16),
            pltpu.VMEM((Bx, hidden), jnp.bfloat16),
        ],
    )(x, base_vectors, dec_w, importance.reshape(1, Bx), dec_b.reshape(1, hidden))

    return out


# speedup vs baseline: 1.8785x; 1.0515x over previous
import jax
import jax.numpy as jnp
from jax.experimental import pallas as pl
from jax.experimental.pallas import tpu as pltpu

_BQ = 512


def _fused_body(x_ref, bv_ref, dw_ref, imp_ref, db_ref, out_ref, en_sc, v2_sc):
    i = pl.program_id(0)
    nblk = pl.num_programs(0) // 2

    @pl.when(i < nblk)
    def _encode():
        e = jnp.dot(x_ref[:, 0, :], bv_ref[...], preferred_element_type=jnp.float32)
        norm = jnp.sqrt(jnp.sum(e * e, axis=-1, keepdims=True))
        en_sc[pl.ds(i * _BQ, _BQ), :] = (e / jnp.maximum(norm, 1e-8)).astype(jnp.bfloat16)
        v2_sc[pl.ds(i * _BQ, _BQ), :] = jax.lax.dot_general(
            e, dw_ref[...],
            dimension_numbers=(((1,), (1,)), ((), ())),
            preferred_element_type=jnp.float32,
        ).astype(jnp.bfloat16)

    @pl.when(i >= nblk)
    def _attend():
        j = i - nblk
        enq = en_sc[pl.ds(j * _BQ, _BQ), :]
        s = jax.lax.dot_general(
            enq, en_sc[...],
            dimension_numbers=(((1,), (1,)), ((), ())),
            preferred_element_type=jnp.float32,
        )
        p = jnp.exp(s * imp_ref[...])
        r = jnp.dot(p.astype(jnp.bfloat16), v2_sc[...], preferred_element_type=jnp.float32)
        denom = jnp.sum(p, axis=-1, keepdims=True)
        out_ref[:, 0, :] = r / denom + db_ref[...] + x_ref[:, 0, :]


def kernel(x, importance, base_vectors, dec_w, dec_b, memory_storage, memory_importance):
    Bx = x.shape[0]
    hidden = x.shape[2]
    hd = base_vectors.shape[1]
    nblk = Bx // _BQ

    out = pl.pallas_call(
        _fused_body,
        grid=(2 * nblk,),
        in_specs=[
            pl.BlockSpec((_BQ, 1, hidden), lambda i: (i % (pl.num_programs(0) // 2), 0, 0)),
            pl.BlockSpec((hidden, hd), lambda i: (0, 0)),
            pl.BlockSpec((hidden, hd), lambda i: (0, 0)),
            pl.BlockSpec((1, Bx), lambda i: (0, 0)),
            pl.BlockSpec((1, hidden), lambda i: (0, 0)),
        ],
        out_specs=pl.BlockSpec(
            (_BQ, 1, hidden),
            lambda i: (jnp.maximum(i - pl.num_programs(0) // 2, 0), 0, 0),
        ),
        out_shape=jax.ShapeDtypeStruct((Bx, 1, hidden), jnp.float32),
        scratch_shapes=[
            pltpu.VMEM((Bx, hd), jnp.bfloat16),
            pltpu.VMEM((Bx, hidden), jnp.bfloat16),
        ],
    )(x, base_vectors, dec_w, importance.reshape(1, Bx), dec_b.reshape(1, hidden))

    return out


# speedup vs baseline: 1.9121x; 1.0179x over previous
import jax
import jax.numpy as jnp
from jax.experimental import pallas as pl
from jax.experimental.pallas import tpu as pltpu

_BQ = 1024


def _fused_body(x_ref, bv_ref, dw_ref, imp_ref, db_ref, out_ref, en_sc, v2_sc):
    i = pl.program_id(0)
    nblk = pl.num_programs(0) // 2

    @pl.when(i < nblk)
    def _encode():
        e = jnp.dot(x_ref[:, 0, :], bv_ref[...], preferred_element_type=jnp.float32)
        norm = jnp.sqrt(jnp.sum(e * e, axis=-1, keepdims=True))
        en_sc[pl.ds(i * _BQ, _BQ), :] = (e / jnp.maximum(norm, 1e-8)).astype(jnp.bfloat16)
        v2_sc[pl.ds(i * _BQ, _BQ), :] = jax.lax.dot_general(
            e, dw_ref[...],
            dimension_numbers=(((1,), (1,)), ((), ())),
            preferred_element_type=jnp.float32,
        ).astype(jnp.bfloat16)

    @pl.when(i >= nblk)
    def _attend():
        j = i - nblk
        enq = en_sc[pl.ds(j * _BQ, _BQ), :]
        s = jax.lax.dot_general(
            enq, en_sc[...],
            dimension_numbers=(((1,), (1,)), ((), ())),
            preferred_element_type=jnp.float32,
        )
        p = jnp.exp(s * imp_ref[...])
        r = jnp.dot(p.astype(jnp.bfloat16), v2_sc[...], preferred_element_type=jnp.float32)
        denom = jnp.sum(p, axis=-1, keepdims=True)
        out_ref[:, 0, :] = r / denom + db_ref[...] + x_ref[:, 0, :]


def kernel(x, importance, base_vectors, dec_w, dec_b, memory_storage, memory_importance):
    Bx = x.shape[0]
    hidden = x.shape[2]
    hd = base_vectors.shape[1]
    nblk = Bx // _BQ

    out = pl.pallas_call(
        _fused_body,
        grid=(2 * nblk,),
        in_specs=[
            pl.BlockSpec((_BQ, 1, hidden), lambda i: (i % (pl.num_programs(0) // 2), 0, 0)),
            pl.BlockSpec((hidden, hd), lambda i: (0, 0)),
            pl.BlockSpec((hidden, hd), lambda i: (0, 0)),
            pl.BlockSpec((1, Bx), lambda i: (0, 0)),
            pl.BlockSpec((1, hidden), lambda i: (0, 0)),
        ],
        out_specs=pl.BlockSpec(
            (_BQ, 1, hidden),
            lambda i: (jnp.maximum(i - pl.num_programs(0) // 2, 0), 0, 0),
        ),
        out_shape=jax.ShapeDtypeStruct((Bx, 1, hidden), jnp.float32),
        scratch_shapes=[
            pltpu.VMEM((Bx, hd), jnp.bfloat16),
            pltpu.VMEM((Bx, hidden), jnp.bfloat16),
        ],
    )(x, base_vectors, dec_w, importance.reshape(1, Bx), dec_b.reshape(1, hidden))

    return out
